# Initial kernel scaffold; baseline (speedup 1.0000x reference)
import jax
import jax.numpy as jnp
from jax.experimental import pallas as pl


def kernel(mem, idx, val):
    raise NotImplementedError("write your pallas kernel here")


# speedup vs baseline: 17.2896x; 17.2896x over previous
import functools

import jax
import jax.numpy as jnp
from jax import lax
from jax.experimental import pallas as pl
from jax.experimental.pallas import tpu as pltpu
from jax.experimental.pallas import tpu_sc as plsc

_M = 65536
_D = 256
_B = 4096
_L = 16
_NC = 2
_NS = 16
_NW = _NC * _NS
_RPW = _B // _NW
_CH = _B // _L
_JBITS = 12


def _sc_body(idx_hbm, val_hbm, out_hbm, idx_v, tbl_v, nxt_v, w_v, rows_v, sem):
    wid = lax.axis_index("s") * _NC + lax.axis_index("c")
    io = lax.iota(jnp.int32, _L)

    pltpu.sync_copy(idx_hbm, idx_v)

    def chunk(c, carry):
        base = pl.multiple_of(c * _L, _L)
        iv = idx_v[pl.ds(base, _L)]
        jv = c * _L + io
        ks, js = plsc.sort_key_val((iv << _JBITS) | jv, jv)
        slots = ks >> _JBITS
        nxt_v[...] = slots
        nxt = plsc.load_gather(nxt_v, [jnp.minimum(io + 1, _L - 1)])
        last_of_run = (slots != nxt) | (io == _L - 1)
        plsc.store_scatter(tbl_v, [slots], js, mask=last_of_run)
        return carry

    lax.fori_loop(0, _CH, chunk, 0)

    for c2 in range(_RPW // _L):
        base = pl.multiple_of(wid * _RPW + c2 * _L, _L)
        iv = idx_v[pl.ds(base, _L)]
        w_v[pl.ds(c2 * _L, _L)] = plsc.load_gather(tbl_v, [iv])

    pltpu.async_copy(val_hbm.at[w_v], rows_v, sem).wait()
    pltpu.sync_copy(rows_v, out_hbm.at[pl.ds(wid * _RPW, _RPW)])


_sc_kernel = functools.partial(
    pl.kernel,
    out_type=jax.ShapeDtypeStruct((_B, _D), jnp.float32),
    mesh=plsc.VectorSubcoreMesh(
        core_axis_name="c", subcore_axis_name="s",
        num_cores=_NC, num_subcores=_NS,
    ),
    compiler_params=pltpu.CompilerParams(needs_layout_passes=False),
    scratch_types=[
        pltpu.VMEM((_B,), jnp.int32),
        pltpu.VMEM((_M,), jnp.int32),
        pltpu.VMEM((_L,), jnp.int32),
        pltpu.VMEM((_RPW,), jnp.int32),
        pltpu.VMEM((_RPW, _D), jnp.float32),
        pltpu.SemaphoreType.DMA,
    ],
)(_sc_body)


def kernel(mem, idx, val):
    del mem
    return _sc_kernel(idx.astype(jnp.int32), val)


# speedup vs baseline: 18.6006x; 1.0758x over previous
import functools

import jax
import jax.numpy as jnp
from jax import lax
from jax.experimental import pallas as pl
from jax.experimental.pallas import tpu as pltpu
from jax.experimental.pallas import tpu_sc as plsc

_M = 65536
_D = 256
_B = 4096
_L = 16
_NC = 2
_NS = 16
_NW = _NC * _NS
_RPW = _B // _NW
_CH = _B // _L
_JBITS = 12


def _sc_body(idx_hbm, val_hbm, out_hbm, idx_v, tbl_v, nxt_v, w_v, rows_v, sem):
    wid = lax.axis_index("s") * _NC + lax.axis_index("c")
    io = lax.iota(jnp.int32, _L)

    pltpu.sync_copy(idx_hbm, idx_v)

    def chunk(c, carry):
        base = pl.multiple_of(c * _L, _L)
        iv = idx_v[pl.ds(base, _L)]
        _, last = plsc.scan_count(iv)
        plsc.store_scatter(tbl_v, [iv], c * _L + io, mask=last)
        return carry

    lax.fori_loop(0, _CH, chunk, 0, unroll=4)

    for c2 in range(_RPW // _L):
        base = pl.multiple_of(wid * _RPW + c2 * _L, _L)
        iv = idx_v[pl.ds(base, _L)]
        w_v[pl.ds(c2 * _L, _L)] = plsc.load_gather(tbl_v, [iv])

    pltpu.async_copy(val_hbm.at[w_v], rows_v, sem).wait()
    pltpu.sync_copy(rows_v, out_hbm.at[pl.ds(wid * _RPW, _RPW)])


_sc_kernel = functools.partial(
    pl.kernel,
    out_type=jax.ShapeDtypeStruct((_B, _D), jnp.float32),
    mesh=plsc.VectorSubcoreMesh(
        core_axis_name="c", subcore_axis_name="s",
        num_cores=_NC, num_subcores=_NS,
    ),
    compiler_params=pltpu.CompilerParams(needs_layout_passes=False),
    scratch_types=[
        pltpu.VMEM((_B,), jnp.int32),
        pltpu.VMEM((_M,), jnp.int32),
        pltpu.VMEM((_L,), jnp.int32),
        pltpu.VMEM((_RPW,), jnp.int32),
        pltpu.VMEM((_RPW, _D), jnp.float32),
        pltpu.SemaphoreType.DMA,
    ],
)(_sc_body)


def kernel(mem, idx, val):
    del mem
    return _sc_kernel(idx.astype(jnp.int32), val)


# speedup vs baseline: 19.9341x; 1.0717x over previous
import functools

import jax
import jax.numpy as jnp
from jax import lax
from jax.experimental import pallas as pl
from jax.experimental.pallas import tpu as pltpu
from jax.experimental.pallas import tpu_sc as plsc

_M = 65536
_D = 256
_B = 4096
_L = 16
_NC = 2
_NS = 16
_NW = _NC * _NS
_RPW = _B // _NW
_CH = _B // _L
_NSEG = 4
_NB = 2
_RB = _RPW // _NB


def _sc_body(idx_hbm, val_hbm, out_hbm, idx_v, tbl_v, w_v, rows_v,
             gsem, wsem, isem):
    wid = lax.axis_index("s") * _NC + lax.axis_index("c")
    io = lax.iota(jnp.int32, _L)

    seg = _B // _NSEG
    copies = [
        pltpu.async_copy(idx_hbm.at[pl.ds(s * seg, seg)],
                         idx_v.at[pl.ds(s * seg, seg)], isem)
        for s in range(_NSEG)
    ]

    def scan_range(lo, hi):
        iv0 = idx_v[pl.ds(pl.multiple_of(lo * _L, _L), _L)]
        _, last0 = plsc.scan_count(iv0)

        def chunk(c, carry):
            iv_c, last_c = carry
            nbase = pl.multiple_of(jnp.minimum(c + 1, hi - 1) * _L, _L)
            iv_n = idx_v[pl.ds(nbase, _L)]
            _, last_n = plsc.scan_count(iv_n)
            plsc.store_scatter(tbl_v, [iv_c], c * _L + io, mask=last_c)
            return iv_n, last_n

        lax.fori_loop(lo, hi, chunk, (iv0, last0), unroll=2)

    for s in range(_NSEG):
        copies[s].wait()
        scan_range(s * (_CH // _NSEG), (s + 1) * (_CH // _NSEG))

    for c2 in range(_RPW // _L):
        base = pl.multiple_of(wid * _RPW + c2 * _L, _L)
        iv = idx_v[pl.ds(base, _L)]
        w_v[pl.ds(c2 * _L, _L)] = plsc.load_gather(tbl_v, [iv])

    writes = []
    g_prev = pltpu.async_copy(
        val_hbm.at[w_v.at[pl.ds(0, _RB)]], rows_v.at[pl.ds(0, _RB)], gsem)
    for k in range(_NB):
        if k + 1 < _NB:
            g_next = pltpu.async_copy(
                val_hbm.at[w_v.at[pl.ds((k + 1) * _RB, _RB)]],
                rows_v.at[pl.ds((k + 1) * _RB, _RB)], gsem)
        g_prev.wait()
        writes.append(pltpu.async_copy(
            rows_v.at[pl.ds(k * _RB, _RB)],
            out_hbm.at[pl.ds(wid * _RPW + k * _RB, _RB)], wsem))
        if k + 1 < _NB:
            g_prev = g_next
    for wcp in writes:
        wcp.wait()


_sc_kernel = functools.partial(
    pl.kernel,
    out_type=jax.ShapeDtypeStruct((_B, _D), jnp.float32),
    mesh=plsc.VectorSubcoreMesh(
        core_axis_name="c", subcore_axis_name="s",
        num_cores=_NC, num_subcores=_NS,
    ),
    compiler_params=pltpu.CompilerParams(needs_layout_passes=False),
    scratch_types=[
        pltpu.VMEM((_B,), jnp.int32),
        pltpu.VMEM((_M,), jnp.int32),
        pltpu.VMEM((_RPW,), jnp.int32),
        pltpu.VMEM((_RPW, _D), jnp.float32),
        pltpu.SemaphoreType.DMA,
        pltpu.SemaphoreType.DMA,
        pltpu.SemaphoreType.DMA,
    ],
)(_sc_body)


def kernel(mem, idx, val):
    del mem
    return _sc_kernel(idx.astype(jnp.int32), val)


# speedup vs baseline: 20.0876x; 1.0077x over previous
import functools

import jax
import jax.numpy as jnp
from jax import lax
from jax.experimental import pallas as pl
from jax.experimental.pallas import tpu as pltpu
from jax.experimental.pallas import tpu_sc as plsc

_M = 65536
_D = 256
_B = 4096
_L = 16
_NC = 2
_NS = 16
_NW = _NC * _NS
_RPW = _B // _NW
_CH = _B // _L
_NSEG = 4
_NB = 2
_RB = _RPW // _NB


def _sc_body(idx_hbm, val_hbm, out_hbm, idx_v, tbl_v, w_v, rows_v,
             gsem, wsem, isem):
    wid = lax.axis_index("s") * _NC + lax.axis_index("c")
    io = lax.iota(jnp.int32, _L)

    seg = _B // _NSEG
    copies = [
        pltpu.async_copy(idx_hbm.at[pl.ds(s * seg, seg)],
                         idx_v.at[pl.ds(s * seg, seg)], isem)
        for s in range(_NSEG)
    ]

    def scan_range(lo, hi):
        iv0 = idx_v[pl.ds(pl.multiple_of(lo * _L, _L), _L)]
        _, l0 = plsc.scan_count(iv0)
        iv1 = idx_v[pl.ds(pl.multiple_of((lo + 1) * _L, _L), _L)]
        _, l1 = plsc.scan_count(iv1)

        def chunk(c, carry):
            iv_a, l_a, iv_b, l_b = carry
            nbase = pl.multiple_of(jnp.minimum(c + 2, hi - 1) * _L, _L)
            iv_n = idx_v[pl.ds(nbase, _L)]
            _, l_n = plsc.scan_count(iv_n)
            plsc.store_scatter(tbl_v, [iv_a], c * _L + io, mask=l_a)
            return iv_b, l_b, iv_n, l_n

        lax.fori_loop(lo, hi, chunk, (iv0, l0, iv1, l1), unroll=4)

    for s in range(_NSEG):
        copies[s].wait()
        scan_range(s * (_CH // _NSEG), (s + 1) * (_CH // _NSEG))

    for c2 in range(_RPW // _L):
        base = pl.multiple_of(wid * _RPW + c2 * _L, _L)
        iv = idx_v[pl.ds(base, _L)]
        w_v[pl.ds(c2 * _L, _L)] = plsc.load_gather(tbl_v, [iv])

    writes = []
    g_prev = pltpu.async_copy(
        val_hbm.at[w_v.at[pl.ds(0, _RB)]], rows_v.at[pl.ds(0, _RB)], gsem)
    for k in range(_NB):
        if k + 1 < _NB:
            g_next = pltpu.async_copy(
                val_hbm.at[w_v.at[pl.ds((k + 1) * _RB, _RB)]],
                rows_v.at[pl.ds((k + 1) * _RB, _RB)], gsem)
        g_prev.wait()
        writes.append(pltpu.async_copy(
            rows_v.at[pl.ds(k * _RB, _RB)],
            out_hbm.at[pl.ds(wid * _RPW + k * _RB, _RB)], wsem))
        if k + 1 < _NB:
            g_prev = g_next
    for wcp in writes:
        wcp.wait()


_sc_kernel = functools.partial(
    pl.kernel,
    out_type=jax.ShapeDtypeStruct((_B, _D), jnp.float32),
    mesh=plsc.VectorSubcoreMesh(
        core_axis_name="c", subcore_axis_name="s",
        num_cores=_NC, num_subcores=_NS,
    ),
    compiler_params=pltpu.CompilerParams(needs_layout_passes=False),
    scratch_types=[
        pltpu.VMEM((_B,), jnp.int32),
        pltpu.VMEM((_M,), jnp.int32),
        pltpu.VMEM((_RPW,), jnp.int32),
        pltpu.VMEM((_RPW, _D), jnp.float32),
        pltpu.SemaphoreType.DMA,
        pltpu.SemaphoreType.DMA,
        pltpu.SemaphoreType.DMA,
    ],
)(_sc_body)


def kernel(mem, idx, val):
    del mem
    return _sc_kernel(idx.astype(jnp.int32), val)
